# baseline (speedup 1.0000x reference)
import jax
import jax.numpy as jnp
from jax import lax
from jax.experimental import pallas as pl
from jax.experimental.pallas import tpu as pltpu
from jax.experimental.pallas import tpu_sc as plsc

NC = 2
NS = 16
L = 16
NW = NC * NS

BATCH = 16384
F = 32
B_PER_W = BATCH // NW
CHUNK = 128
N_CHUNKS = B_PER_W // CHUNK


def _sc_body(user_h, item_h, uf_h, if_h, b_h, bu_h, bi_h, out_h,
             idx_u, idx_i, u_rows, v_rows, bu_v, bi_v, out_v, b_v,
             sem_rows, sem_bias):
    wid = lax.axis_index("s") * NC + lax.axis_index("c")

    pltpu.sync_copy(user_h.at[pl.ds(wid * N_CHUNKS, N_CHUNKS)], idx_u)
    pltpu.sync_copy(item_h.at[pl.ds(wid * N_CHUNKS, N_CHUNKS)], idx_i)
    pltpu.sync_copy(b_h, b_v)

    copies = []
    for c in range(N_CHUNKS):
        sl = pl.ds(c * CHUNK, CHUNK)
        copies.append(pltpu.async_copy(
            uf_h.at[idx_u.at[c]], u_rows.at[sl], sem_rows))
        copies.append(pltpu.async_copy(
            if_h.at[idx_i.at[c]], v_rows.at[sl], sem_rows))
        copies.append(pltpu.async_copy(
            bu_h.at[idx_u.at[c]], bu_v.at[sl], sem_bias))
        copies.append(pltpu.async_copy(
            bi_h.at[idx_i.at[c]], bi_v.at[sl], sem_bias))
    for cp in copies:
        cp.wait()

    lane = lax.iota(jnp.int32, L)
    b_vec = b_v[...]

    def blk_body(blk, _):
        rbase = blk * L
        rows = rbase + lane
        acc = jnp.zeros((L,), jnp.float32)
        for f in range(F):
            cols = jnp.full((L,), f, jnp.int32)
            uvals = plsc.load_gather(u_rows, [rows, cols])
            vvals = plsc.load_gather(v_rows, [rows, cols])
            acc = acc + uvals * vvals
        bu = bu_v[pl.ds(rbase, L)]
        bi = bi_v[pl.ds(rbase, L)]
        out_v[pl.ds(rbase, L)] = acc + b_vec + bu + bi
        return 0

    lax.fori_loop(0, B_PER_W // L, blk_body, 0)

    pltpu.sync_copy(out_v, out_h.at[pl.ds(wid * B_PER_W, B_PER_W)])


def kernel(user, item, user_factors, item_factors, b, b_u, b_i):
    b16 = jnp.broadcast_to(b, (L,))
    bu_flat = b_u.reshape(-1)
    bi_flat = b_i.reshape(-1)
    user2d = user.reshape(NW * N_CHUNKS, CHUNK)
    item2d = item.reshape(NW * N_CHUNKS, CHUNK)
    mesh = plsc.VectorSubcoreMesh(core_axis_name="c", subcore_axis_name="s")
    f = pl.kernel(
        _sc_body,
        out_type=jax.ShapeDtypeStruct((BATCH,), jnp.float32),
        mesh=mesh,
        scratch_types=[
            pltpu.VMEM((N_CHUNKS, CHUNK), jnp.int32),
            pltpu.VMEM((N_CHUNKS, CHUNK), jnp.int32),
            pltpu.VMEM((B_PER_W, F), jnp.float32),
            pltpu.VMEM((B_PER_W, F), jnp.float32),
            pltpu.VMEM((B_PER_W,), jnp.float32),
            pltpu.VMEM((B_PER_W,), jnp.float32),
            pltpu.VMEM((B_PER_W,), jnp.float32),
            pltpu.VMEM((L,), jnp.float32),
            pltpu.SemaphoreType.DMA,
            pltpu.SemaphoreType.DMA,
        ],
        compiler_params=pltpu.CompilerParams(
            use_tc_tiling_on_sc=False, needs_layout_passes=False),
    )
    return f(user2d, item2d, user_factors, item_factors, b16, bu_flat, bi_flat)


# speedup vs baseline: 2.9183x; 2.9183x over previous
import jax
import jax.numpy as jnp
from jax import lax
from jax.experimental import pallas as pl
from jax.experimental.pallas import tpu as pltpu
from jax.experimental.pallas import tpu_sc as plsc

NC = 2
NS = 16
L = 16
NW = NC * NS

BATCH = 16384
F = 32
B_PER_W = BATCH // NW
CHUNK = 128
N_CHUNKS = B_PER_W // CHUNK

NBLK = 7812
BLK_PER_W = 245
WIN_BLK = 8
WIN_U = WIN_BLK * 128
TAIL0 = NBLK * 128
ROW_PAD = BATCH + L
HIT_CAP = BATCH + L


def _scan_body(user2, item2, ufT, ifT, uo_h, vo_h,
               idx_all, hits, wl, wbuf, stages, sems, wsem2):
    wid = lax.axis_index("s") * NC + lax.axis_index("c")
    lo = wid * BLK_PER_W
    hi = jnp.minimum(NBLK, lo + BLK_PER_W)
    lo_u = lo * 128
    hi_u = hi * 128
    nwin = (hi - lo + WIN_BLK - 1) // WIN_BLK
    lane = lax.iota(jnp.int32, L)

    def phase(idx_h, table_h, out_h):
        pltpu.sync_copy(idx_h, idx_all)

        def scan_body(j, cnt):
            u = idx_all[j >> 3, pl.ds((j & 7) * 16, 16)]
            m = (u >= lo_u) & (u < hi_u)
            pc = plsc.all_reduce_population_count(m)
            packed = ((u - lo_u) << 14) + (j * 16 + lane)
            plsc.store_compressed(hits.at[pl.ds(cnt, 16)], packed, mask=m)
            return cnt + jnp.max(pc)

        cnt = lax.fori_loop(0, BATCH // L, scan_body, 0)
        nhv = (cnt + 15) // 16

        def issue(g, p):
            wlo = jnp.minimum(lo + g * WIN_BLK, hi - WIN_BLK)
            for fb in range(4):
                pltpu.async_copy(
                    table_h.at[pl.ds(fb * 8, 8), pl.ds(wlo * 128, WIN_U)],
                    wbuf.at[p].at[fb], wsem2.at[p])

        issue(0, 0)

        def win_body(g, _):
            p = g & 1
            wlo = jnp.minimum(lo + g * WIN_BLK, hi - WIN_BLK)
            wrel = (wlo - lo) * 128
            for fb in range(4):
                pltpu.make_async_copy(
                    table_h.at[pl.ds(fb * 8, 8), pl.ds(0, WIN_U)],
                    wbuf.at[p].at[fb], wsem2.at[p]).wait()

            @pl.when(g + 1 < nwin)
            def _prefetch():
                issue(g + 1, 1 - p)

            def wscan(j, wcnt):
                v = hits[pl.ds(j * 16, 16)]
                ulr = v >> 14
                m = (ulr >= wrel) & (ulr < wrel + WIN_U)
                pc = plsc.all_reduce_population_count(m)
                plsc.store_compressed(wl.at[pl.ds(wcnt, 16)],
                                      v - (wrel << 14), mask=m)
                return wcnt + jnp.max(pc)

            wcnt = lax.fori_loop(0, nhv, wscan, 0)
            ng = (wcnt + 15) // 16

            def grp(k, _):
                r = k & 3

                @pl.when(k >= 4)
                def _wait_slot():
                    pltpu.make_async_copy(
                        stages.at[r], out_h.at[pl.ds(0, 16)],
                        sems.at[r]).wait()

                v = wl[pl.ds(k * 16, 16)]
                ul = jnp.clip(v >> 14, 0, WIN_U - 1)
                sl = v & (BATCH - 1)
                valid = (k * 16 + lane) < wcnt
                slots = jnp.where(valid, sl, BATCH + lane)
                pvec = jnp.full((L,), p, jnp.int32)
                for f in range(F):
                    vals = plsc.load_gather(
                        wbuf, [pvec, jnp.full((L,), f >> 3, jnp.int32),
                               jnp.full((L,), f & 7, jnp.int32), ul])
                    plsc.store_scatter(
                        stages.at[r],
                        [lane, jnp.full((L,), f, jnp.int32)], vals)
                pltpu.async_copy(stages.at[r], out_h.at[slots], sems.at[r])
                return 0

            lax.fori_loop(0, ng, grp, 0)

            def drain(r2, _):
                pltpu.make_async_copy(
                    stages.at[r2], out_h.at[pl.ds(0, 16)],
                    sems.at[r2]).wait()
                return 0

            lax.fori_loop(0, jnp.minimum(ng, 4), drain, 0)
            return 0

        lax.fori_loop(0, nwin, win_body, 0)

    phase(user2, ufT, uo_h)
    phase(item2, ifT, vo_h)


def _dot_body(user2, item2, uo_h, vo_h, b_h, bu_h, bi_h, ut_h, it_h, out_h,
              idx_u, idx_i, uc, vc, bu_v, bi_v, ut_v, it_v, out_v, b_v,
              sem_rows, sem_bias):
    wid = lax.axis_index("s") * NC + lax.axis_index("c")
    base = wid * B_PER_W

    pltpu.sync_copy(user2.at[pl.ds(wid * N_CHUNKS, N_CHUNKS)], idx_u)
    pltpu.sync_copy(item2.at[pl.ds(wid * N_CHUNKS, N_CHUNKS)], idx_i)
    pltpu.sync_copy(b_h, b_v)
    pltpu.sync_copy(ut_h, ut_v)
    pltpu.sync_copy(it_h, it_v)

    copies = []
    for c in range(N_CHUNKS):
        sl = pl.ds(c * CHUNK, CHUNK)
        copies.append(pltpu.async_copy(
            bu_h.at[idx_u.at[c]], bu_v.at[sl], sem_bias))
        copies.append(pltpu.async_copy(
            bi_h.at[idx_i.at[c]], bi_v.at[sl], sem_bias))
    for cp in copies:
        cp.wait()

    lane = lax.iota(jnp.int32, L)
    b_vec = b_v[...]

    def chunk_body(c, _):
        pltpu.sync_copy(uo_h.at[pl.ds(base + c * CHUNK, CHUNK)], uc)
        pltpu.sync_copy(vo_h.at[pl.ds(base + c * CHUNK, CHUNK)], vc)

        def blk_body(blk, _2):
            b_glob = c * (CHUNK // L) + blk
            rbase = blk * L
            rows = rbase + lane
            uvec = idx_u[b_glob >> 3, pl.ds((b_glob & 7) * 16, 16)]
            ivec = idx_i[b_glob >> 3, pl.ds((b_glob & 7) * 16, 16)]
            tm_u = uvec >= TAIL0
            tm_i = ivec >= TAIL0
            tid_u = jnp.clip(uvec - TAIL0, 0, 63)
            tid_i = jnp.clip(ivec - TAIL0, 0, 63)
            acc = jnp.zeros((L,), jnp.float32)
            for f in range(F):
                cols = jnp.full((L,), f, jnp.int32)
                uval = plsc.load_gather(uc, [rows, cols])
                vval = plsc.load_gather(vc, [rows, cols])
                utail = plsc.load_gather(ut_v, [tid_u, cols])
                vtail = plsc.load_gather(it_v, [tid_i, cols])
                uval = jnp.where(tm_u, utail, uval)
                vval = jnp.where(tm_i, vtail, vval)
                acc = acc + uval * vval
            obase = c * CHUNK + rbase
            bu = bu_v[pl.ds(obase, L)]
            bi = bi_v[pl.ds(obase, L)]
            out_v[pl.ds(obase, L)] = acc + b_vec + bu + bi
            return 0

        lax.fori_loop(0, CHUNK // L, blk_body, 0)
        return 0

    lax.fori_loop(0, N_CHUNKS, chunk_body, 0)
    pltpu.sync_copy(out_v, out_h.at[pl.ds(base, B_PER_W)])


def kernel(user, item, user_factors, item_factors, b, b_u, b_i):
    b16 = jnp.broadcast_to(b, (L,))
    bu_flat = b_u.reshape(-1)
    bi_flat = b_i.reshape(-1)
    user2d = user.reshape(NW * N_CHUNKS, CHUNK)
    item2d = item.reshape(NW * N_CHUNKS, CHUNK)
    ufT = user_factors.T
    ifT = item_factors.T
    uf_tail = user_factors[TAIL0:]
    if_tail = item_factors[TAIL0:]
    mesh = plsc.VectorSubcoreMesh(core_axis_name="c", subcore_axis_name="s")

    scan = pl.kernel(
        _scan_body,
        out_type=(jax.ShapeDtypeStruct((ROW_PAD, 128), jnp.float32),
                  jax.ShapeDtypeStruct((ROW_PAD, 128), jnp.float32)),
        mesh=mesh,
        scratch_types=[
            pltpu.VMEM((128, 128), jnp.int32),
            pltpu.VMEM((HIT_CAP,), jnp.int32),
            pltpu.VMEM((HIT_CAP,), jnp.int32),
            pltpu.VMEM((2, 4, 8, WIN_U), jnp.float32),
            pltpu.VMEM((4, L, 128), jnp.float32),
            pltpu.SemaphoreType.DMA((4,)),
            pltpu.SemaphoreType.DMA((2,)),
        ],
        compiler_params=pltpu.CompilerParams(
            use_tc_tiling_on_sc=True, needs_layout_passes=False),
    )
    u_rows, v_rows = scan(user2d, item2d, ufT, ifT)

    dot = pl.kernel(
        _dot_body,
        out_type=jax.ShapeDtypeStruct((BATCH,), jnp.float32),
        mesh=mesh,
        scratch_types=[
            pltpu.VMEM((N_CHUNKS, CHUNK), jnp.int32),
            pltpu.VMEM((N_CHUNKS, CHUNK), jnp.int32),
            pltpu.VMEM((CHUNK, 128), jnp.float32),
            pltpu.VMEM((CHUNK, 128), jnp.float32),
            pltpu.VMEM((B_PER_W,), jnp.float32),
            pltpu.VMEM((B_PER_W,), jnp.float32),
            pltpu.VMEM((64, F), jnp.float32),
            pltpu.VMEM((64, F), jnp.float32),
            pltpu.VMEM((B_PER_W,), jnp.float32),
            pltpu.VMEM((L,), jnp.float32),
            pltpu.SemaphoreType.DMA,
            pltpu.SemaphoreType.DMA,
        ],
        compiler_params=pltpu.CompilerParams(
            use_tc_tiling_on_sc=False, needs_layout_passes=False),
    )
    return dot(user2d, item2d, u_rows, v_rows, b16, bu_flat, bi_flat,
               uf_tail, if_tail)


# speedup vs baseline: 2.9223x; 1.0014x over previous
import jax
import jax.numpy as jnp
from jax import lax
from jax.experimental import pallas as pl
from jax.experimental.pallas import tpu as pltpu
from jax.experimental.pallas import tpu_sc as plsc

NC = 2
NS = 16
L = 16
NW = NC * NS

BATCH = 16384
F = 32
B_PER_W = BATCH // NW
CHUNK = 128
N_CHUNKS = B_PER_W // CHUNK

NBLK = 7812
BLK_PER_W = 245
WIN_BLK = 8
WIN_U = WIN_BLK * 128
TAIL0 = NBLK * 128
ROW_PAD = BATCH + L
HIT_CAP = BATCH + L


def _scan_body(user2, item2, ufT, ifT, uo_h, vo_h,
               idx_all, hits, wl, wbuf, stages, sems, wsem2):
    wid = lax.axis_index("s") * NC + lax.axis_index("c")
    lo = wid * BLK_PER_W
    hi = jnp.minimum(NBLK, lo + BLK_PER_W)
    lo_u = lo * 128
    hi_u = hi * 128
    nwin = (hi - lo + WIN_BLK - 1) // WIN_BLK
    lane = lax.iota(jnp.int32, L)

    def phase(idx_h, table_h, out_h):
        pltpu.sync_copy(idx_h, idx_all)

        def scan_body(j, cnt):
            u = idx_all[j >> 3, pl.ds((j & 7) * 16, 16)]
            m = (u >= lo_u) & (u < hi_u)
            pc = plsc.all_reduce_population_count(m)
            packed = ((u - lo_u) << 14) + (j * 16 + lane)
            plsc.store_compressed(hits.at[pl.ds(cnt, 16)], packed, mask=m)
            return cnt + jnp.max(pc)

        cnt = lax.fori_loop(0, BATCH // L, scan_body, 0)
        nhv = (cnt + 15) // 16

        def issue(g, p):
            wlo = jnp.minimum(lo + g * WIN_BLK, hi - WIN_BLK)
            for fb in range(4):
                pltpu.async_copy(
                    table_h.at[pl.ds(fb * 8, 8), pl.ds(wlo * 128, WIN_U)],
                    wbuf.at[p].at[fb], wsem2.at[p])

        issue(0, 0)

        def win_body(g, _):
            p = g & 1
            wlo = jnp.minimum(lo + g * WIN_BLK, hi - WIN_BLK)
            wrel = (wlo - lo) * 128
            for fb in range(4):
                pltpu.make_async_copy(
                    table_h.at[pl.ds(fb * 8, 8), pl.ds(0, WIN_U)],
                    wbuf.at[p].at[fb], wsem2.at[p]).wait()

            @pl.when(g + 1 < nwin)
            def _prefetch():
                issue(g + 1, 1 - p)

            def wscan(j, wcnt):
                v = hits[pl.ds(j * 16, 16)]
                ulr = v >> 14
                m = (ulr >= wrel) & (ulr < wrel + WIN_U)
                pc = plsc.all_reduce_population_count(m)
                plsc.store_compressed(wl.at[pl.ds(wcnt, 16)],
                                      v - (wrel << 14), mask=m)
                return wcnt + jnp.max(pc)

            wcnt = lax.fori_loop(0, nhv, wscan, 0)
            ng = (wcnt + 15) // 16

            def grp(k, _):
                r = k & 3

                @pl.when(k >= 4)
                def _wait_slot():
                    pltpu.make_async_copy(
                        stages.at[r], out_h.at[pl.ds(0, 16)],
                        sems.at[r]).wait()

                v = wl[pl.ds(k * 16, 16)]
                ul = jnp.clip(v >> 14, 0, WIN_U - 1)
                sl = v & (BATCH - 1)
                valid = (k * 16 + lane) < wcnt
                slots = jnp.where(valid, sl, BATCH + lane)
                pvec = jnp.full((L,), p, jnp.int32)
                for f in range(F):
                    vals = plsc.load_gather(
                        wbuf, [pvec, jnp.full((L,), f >> 3, jnp.int32),
                               jnp.full((L,), f & 7, jnp.int32), ul])
                    plsc.store_scatter(
                        stages.at[r],
                        [lane, jnp.full((L,), f, jnp.int32)], vals)
                rb = plsc.load_gather(
                    stages.at[r], [lane, jnp.full((L,), F - 1, jnp.int32)])
                slots = jnp.where(rb == rb, slots, lane)
                pltpu.async_copy(stages.at[r], out_h.at[slots], sems.at[r])
                return 0

            lax.fori_loop(0, ng, grp, 0)

            def drain(r2, _):
                pltpu.make_async_copy(
                    stages.at[r2], out_h.at[pl.ds(0, 16)],
                    sems.at[r2]).wait()
                return 0

            lax.fori_loop(0, jnp.minimum(ng, 4), drain, 0)
            return 0

        lax.fori_loop(0, nwin, win_body, 0)

    phase(user2, ufT, uo_h)
    phase(item2, ifT, vo_h)


def _dot_body(user2, item2, uo_h, vo_h, b_h, bu_h, bi_h, ut_h, it_h, out_h,
              idx_u, idx_i, uc, vc, bu_v, bi_v, ut_v, it_v, out_v, b_v,
              sem_rows, sem_bias):
    wid = lax.axis_index("s") * NC + lax.axis_index("c")
    base = wid * B_PER_W

    pltpu.sync_copy(user2.at[pl.ds(wid * N_CHUNKS, N_CHUNKS)], idx_u)
    pltpu.sync_copy(item2.at[pl.ds(wid * N_CHUNKS, N_CHUNKS)], idx_i)
    pltpu.sync_copy(b_h, b_v)
    pltpu.sync_copy(ut_h, ut_v)
    pltpu.sync_copy(it_h, it_v)

    copies = []
    for c in range(N_CHUNKS):
        sl = pl.ds(c * CHUNK, CHUNK)
        copies.append(pltpu.async_copy(
            bu_h.at[idx_u.at[c]], bu_v.at[sl], sem_bias))
        copies.append(pltpu.async_copy(
            bi_h.at[idx_i.at[c]], bi_v.at[sl], sem_bias))
    for cp in copies:
        cp.wait()

    lane = lax.iota(jnp.int32, L)
    b_vec = b_v[...]

    def chunk_body(c, _):
        pltpu.sync_copy(uo_h.at[pl.ds(base + c * CHUNK, CHUNK)], uc)
        pltpu.sync_copy(vo_h.at[pl.ds(base + c * CHUNK, CHUNK)], vc)

        def blk_body(blk, _2):
            b_glob = c * (CHUNK // L) + blk
            rbase = blk * L
            rows = rbase + lane
            uvec = idx_u[b_glob >> 3, pl.ds((b_glob & 7) * 16, 16)]
            ivec = idx_i[b_glob >> 3, pl.ds((b_glob & 7) * 16, 16)]
            tm_u = uvec >= TAIL0
            tm_i = ivec >= TAIL0
            tid_u = jnp.clip(uvec - TAIL0, 0, 63)
            tid_i = jnp.clip(ivec - TAIL0, 0, 63)
            acc = jnp.zeros((L,), jnp.float32)
            for f in range(F):
                cols = jnp.full((L,), f, jnp.int32)
                uval = plsc.load_gather(uc, [rows, cols])
                vval = plsc.load_gather(vc, [rows, cols])
                utail = plsc.load_gather(ut_v, [tid_u, cols])
                vtail = plsc.load_gather(it_v, [tid_i, cols])
                uval = jnp.where(tm_u, utail, uval)
                vval = jnp.where(tm_i, vtail, vval)
                acc = acc + uval * vval
            obase = c * CHUNK + rbase
            bu = bu_v[pl.ds(obase, L)]
            bi = bi_v[pl.ds(obase, L)]
            out_v[pl.ds(obase, L)] = acc + b_vec + bu + bi
            return 0

        lax.fori_loop(0, CHUNK // L, blk_body, 0)
        return 0

    lax.fori_loop(0, N_CHUNKS, chunk_body, 0)
    pltpu.sync_copy(out_v, out_h.at[pl.ds(base, B_PER_W)])


def kernel(user, item, user_factors, item_factors, b, b_u, b_i):
    b16 = jnp.broadcast_to(b, (L,))
    bu_flat = b_u.reshape(-1)
    bi_flat = b_i.reshape(-1)
    user2d = user.reshape(NW * N_CHUNKS, CHUNK)
    item2d = item.reshape(NW * N_CHUNKS, CHUNK)
    ufT = user_factors.T
    ifT = item_factors.T
    uf_tail = user_factors[TAIL0:]
    if_tail = item_factors[TAIL0:]
    mesh = plsc.VectorSubcoreMesh(core_axis_name="c", subcore_axis_name="s")

    scan = pl.kernel(
        _scan_body,
        out_type=(jax.ShapeDtypeStruct((ROW_PAD, 128), jnp.float32),
                  jax.ShapeDtypeStruct((ROW_PAD, 128), jnp.float32)),
        mesh=mesh,
        scratch_types=[
            pltpu.VMEM((128, 128), jnp.int32),
            pltpu.VMEM((HIT_CAP,), jnp.int32),
            pltpu.VMEM((HIT_CAP,), jnp.int32),
            pltpu.VMEM((2, 4, 8, WIN_U), jnp.float32),
            pltpu.VMEM((4, L, 128), jnp.float32),
            pltpu.SemaphoreType.DMA((4,)),
            pltpu.SemaphoreType.DMA((2,)),
        ],
        compiler_params=pltpu.CompilerParams(
            use_tc_tiling_on_sc=True, needs_layout_passes=False),
    )
    u_rows, v_rows = scan(user2d, item2d, ufT, ifT)

    dot = pl.kernel(
        _dot_body,
        out_type=jax.ShapeDtypeStruct((BATCH,), jnp.float32),
        mesh=mesh,
        scratch_types=[
            pltpu.VMEM((N_CHUNKS, CHUNK), jnp.int32),
            pltpu.VMEM((N_CHUNKS, CHUNK), jnp.int32),
            pltpu.VMEM((CHUNK, 128), jnp.float32),
            pltpu.VMEM((CHUNK, 128), jnp.float32),
            pltpu.VMEM((B_PER_W,), jnp.float32),
            pltpu.VMEM((B_PER_W,), jnp.float32),
            pltpu.VMEM((64, F), jnp.float32),
            pltpu.VMEM((64, F), jnp.float32),
            pltpu.VMEM((B_PER_W,), jnp.float32),
            pltpu.VMEM((L,), jnp.float32),
            pltpu.SemaphoreType.DMA,
            pltpu.SemaphoreType.DMA,
        ],
        compiler_params=pltpu.CompilerParams(
            use_tc_tiling_on_sc=False, needs_layout_passes=False),
    )
    return dot(user2d, item2d, u_rows, v_rows, b16, bu_flat, bi_flat,
               uf_tail, if_tail)
